# Initial kernel scaffold; baseline (speedup 1.0000x reference)
import jax
import jax.numpy as jnp
from jax.experimental import pallas as pl


def kernel(tokens, codebook):
    raise NotImplementedError("write your pallas kernel here")


# speedup vs baseline: 8.8492x; 8.8492x over previous
import functools

import jax
import jax.numpy as jnp
from jax import lax
from jax.experimental import pallas as pl
from jax.experimental.pallas import tpu as pltpu
from jax.experimental.pallas import tpu_sc as plsc

_info = plsc.get_sparse_core_info()
_NC, _NS, _L = _info.num_cores, _info.num_subcores, _info.num_lanes
_NW = _NC * _NS

_B, _T = 4096, 200
_N_TOK = _B * _T
_D = 64
_C = _N_TOK // _NW
_G = 256
_NCHUNK = _C // _G
_NP = _NCHUNK // 2


def _body(tok_hbm, tab_hbm, out_hbm, tok_v,
          ilo0, ihi0, lo0, hi0, ilo1, ihi1, lo1, hi1,
          sin0, sin1, sout0, sout1):
    wid = lax.axis_index("s") * _NC + lax.axis_index("c")
    base = wid * _C
    pltpu.sync_copy(tok_hbm.at[pl.ds(base, _C)], tok_v)

    bufs = ((ilo0, ihi0, lo0, hi0, sin0, sout0),
            (ilo1, ihi1, lo1, hi1, sin1, sout1))

    def compute_idx(g, b):
        ilo, ihi = bufs[b][0], bufs[b][1]
        off = g * _G

        def grp(j, c):
            t = tok_v[pl.ds(off + j * _L, _L)]
            ilo[pl.ds(j * _L, _L)] = lax.bitwise_and(t, 127)
            ihi[pl.ds(j * _L, _L)] = lax.shift_right_logical(t, 7) + 128
            return c

        lax.fori_loop(0, _G // _L, grp, 0, unroll=True)

    def fire_gather(b):
        ilo, ihi, lo, hi, sin, _ = bufs[b]
        pltpu.make_async_copy(tab_hbm.at[ilo], lo, sin).start()
        pltpu.make_async_copy(tab_hbm.at[ihi], hi, sin).start()

    def wait_gather(b):
        ilo, ihi, lo, hi, sin, _ = bufs[b]
        pltpu.make_async_copy(tab_hbm.at[ilo], lo, sin).wait()
        pltpu.make_async_copy(tab_hbm.at[ihi], hi, sin).wait()

    def fire_out(g, b):
        lo, hi, sout = bufs[b][2], bufs[b][3], bufs[b][5]
        row0 = base + g * _G
        pltpu.make_async_copy(lo, out_hbm.at[pl.ds(row0, _G), 0], sout).start()
        pltpu.make_async_copy(hi, out_hbm.at[pl.ds(row0, _G), 1], sout).start()

    def wait_out(b):
        lo, hi, sout = bufs[b][2], bufs[b][3], bufs[b][5]
        pltpu.make_async_copy(lo, out_hbm.at[pl.ds(base, _G), 0], sout).wait()
        pltpu.make_async_copy(hi, out_hbm.at[pl.ds(base, _G), 1], sout).wait()

    compute_idx(0, 0)
    fire_gather(0)
    compute_idx(1, 1)
    fire_gather(1)

    def step(g, b):
        wait_gather(b)
        fire_out(g, b)

        @pl.when(g + 2 < _NCHUNK)
        def _():
            wait_out(b)
            compute_idx(g + 2, b)
            fire_gather(b)

    def pair(p, carry):
        step(2 * p, 0)
        step(2 * p + 1, 1)
        return carry

    lax.fori_loop(0, _NP, pair, 0)
    wait_out(0)
    wait_out(1)


@functools.partial(
    pl.kernel,
    out_type=jax.ShapeDtypeStruct((_N_TOK, 2, _D), jnp.float32),
    mesh=plsc.VectorSubcoreMesh(core_axis_name="c", subcore_axis_name="s"),
    compiler_params=pltpu.CompilerParams(use_tc_tiling_on_sc=False),
    scratch_types=[
        pltpu.VMEM((_C,), jnp.int32),
        pltpu.VMEM((_G,), jnp.int32),
        pltpu.VMEM((_G,), jnp.int32),
        pltpu.VMEM((_G, _D), jnp.float32),
        pltpu.VMEM((_G, _D), jnp.float32),
        pltpu.VMEM((_G,), jnp.int32),
        pltpu.VMEM((_G,), jnp.int32),
        pltpu.VMEM((_G, _D), jnp.float32),
        pltpu.VMEM((_G, _D), jnp.float32),
        pltpu.SemaphoreType.DMA,
        pltpu.SemaphoreType.DMA,
        pltpu.SemaphoreType.DMA,
        pltpu.SemaphoreType.DMA,
    ],
)
def _lookup(tok_hbm, tab_hbm, out_hbm, *rest):
    _body(tok_hbm, tab_hbm, out_hbm, *rest)


def kernel(tokens, codebook):
    tok = tokens.astype(jnp.int32).reshape(_N_TOK)
    tab = codebook.reshape(2 * 128, _D)
    out3 = _lookup(tok, tab)
    return out3.reshape(_B, _T, 2 * _D)


# speedup vs baseline: 19.6561x; 2.2212x over previous
import functools

import jax
import jax.numpy as jnp
from jax import lax
from jax.experimental import pallas as pl
from jax.experimental.pallas import tpu as pltpu
from jax.experimental.pallas import tpu_sc as plsc

_info = plsc.get_sparse_core_info()
_NC, _NS, _L = _info.num_cores, _info.num_subcores, _info.num_lanes
_NW = _NC * _NS

_B, _T = 4096, 200
_N_TOK = _B * _T
_D = 64
_V = 128 * 128
_C = _N_TOK // _NW
_G = 256
_NCHUNK = _C // _G
_NP = _NCHUNK // 2
_BPT = 128 // _NS


def _body(tok_hbm, tab_hbm, out_hbm, fus_hbm,
          tok_v, tab_v, blk_v, rows0, rows1, sin0, sin1, sout0, sout1):
    cid = lax.axis_index("c")
    sid = lax.axis_index("s")
    wid = sid * _NC + cid
    base = wid * _C
    F = fus_hbm.at[cid]

    pltpu.sync_copy(tab_hbm, tab_v)
    pltpu.sync_copy(tok_hbm.at[pl.ds(base, _C)], tok_v)

    def left_row(r, c):
        for k in range(4):
            blk_v[r, pl.ds(16 * k, 16)] = tab_v[r, pl.ds(16 * k, 16)]
        return c

    lax.fori_loop(0, 128, left_row, 0)

    def build_block(i, carry):
        h = i * _NS + sid
        cks = [tab_v[128 + h, pl.ds(16 * k, 16)] for k in range(4)]

        def right_row(r, c):
            for k in range(4):
                blk_v[r, pl.ds(64 + 16 * k, 16)] = cks[k]
            return c

        lax.fori_loop(0, 128, right_row, 0)
        pltpu.sync_copy(blk_v, F.at[pl.ds(h * 128, 128)])
        return carry

    lax.fori_loop(0, _BPT, build_block, 0)
    plsc.subcore_barrier()

    bufs = ((rows0, sin0, sout0), (rows1, sin1, sout1))

    def fire_gather(g, b):
        rows, sin, _ = bufs[b]
        idx = tok_v.at[pl.ds(g * _G, _G)]
        pltpu.make_async_copy(F.at[idx], rows, sin).start()

    def wait_gather(b):
        rows, sin, _ = bufs[b]
        pltpu.make_async_copy(F.at[tok_v.at[pl.ds(0, _G)]], rows, sin).wait()

    def fire_out(g, b):
        rows, _, sout = bufs[b]
        pltpu.make_async_copy(rows, out_hbm.at[pl.ds(base + g * _G, _G)],
                              sout).start()

    def wait_out(b):
        rows, _, sout = bufs[b]
        pltpu.make_async_copy(rows, out_hbm.at[pl.ds(base, _G)], sout).wait()

    fire_gather(0, 0)
    fire_gather(1, 1)

    def step(g, b):
        wait_gather(b)
        fire_out(g, b)

        @pl.when(g + 2 < _NCHUNK)
        def _():
            wait_out(b)
            fire_gather(g + 2, b)

    def pair(p, carry):
        step(2 * p, 0)
        step(2 * p + 1, 1)
        return carry

    lax.fori_loop(0, _NP, pair, 0)
    wait_out(0)
    wait_out(1)


@functools.partial(
    pl.kernel,
    out_type=(
        jax.ShapeDtypeStruct((_N_TOK, 2 * _D), jnp.float32),
        jax.ShapeDtypeStruct((_NC, _V, 2 * _D), jnp.float32),
    ),
    mesh=plsc.VectorSubcoreMesh(core_axis_name="c", subcore_axis_name="s"),
    compiler_params=pltpu.CompilerParams(use_tc_tiling_on_sc=False),
    scratch_types=[
        pltpu.VMEM((_C,), jnp.int32),
        pltpu.VMEM((256, _D), jnp.float32),
        pltpu.VMEM((128, 2 * _D), jnp.float32),
        pltpu.VMEM((_G, 2 * _D), jnp.float32),
        pltpu.VMEM((_G, 2 * _D), jnp.float32),
        pltpu.SemaphoreType.DMA,
        pltpu.SemaphoreType.DMA,
        pltpu.SemaphoreType.DMA,
        pltpu.SemaphoreType.DMA,
    ],
)
def _lookup(tok_hbm, tab_hbm, out_hbm, fus_hbm, *rest):
    _body(tok_hbm, tab_hbm, out_hbm, fus_hbm, *rest)


def kernel(tokens, codebook):
    tok = tokens.astype(jnp.int32).reshape(_N_TOK)
    tab = codebook.reshape(2 * 128, _D)
    out2, _ = _lookup(tok, tab)
    return out2.reshape(_B, _T, 2 * _D)


# speedup vs baseline: 19.7235x; 1.0034x over previous
import functools

import jax
import jax.numpy as jnp
from jax import lax
from jax.experimental import pallas as pl
from jax.experimental.pallas import tpu as pltpu
from jax.experimental.pallas import tpu_sc as plsc

_info = plsc.get_sparse_core_info()
_NC, _NS, _L = _info.num_cores, _info.num_subcores, _info.num_lanes
_NW = _NC * _NS

_B, _T = 4096, 200
_N_TOK = _B * _T
_D = 64
_V = 128 * 128
_C = _N_TOK // _NW
_G = 128
_NBUF = 4
_NCHUNK = _C // _G
_NSTEP = _NCHUNK // _NBUF
_BPT = 128 // _NS

assert _C % _G == 0 and _NCHUNK % _NBUF == 0


def _body(tok_hbm, tab_hbm, out_hbm, fus_hbm,
          tok_v, tab_v, blk_v, rows, sins, souts):
    cid = lax.axis_index("c")
    sid = lax.axis_index("s")
    wid = sid * _NC + cid
    base = wid * _C
    F = fus_hbm.at[cid]

    pltpu.sync_copy(tab_hbm, tab_v)
    pltpu.sync_copy(tok_hbm.at[pl.ds(base, _C)], tok_v)

    def left_row(r, c):
        for k in range(4):
            blk_v[r, pl.ds(16 * k, 16)] = tab_v[r, pl.ds(16 * k, 16)]
        return c

    lax.fori_loop(0, 128, left_row, 0)

    def build_block(i, carry):
        h = i * _NS + sid
        cks = [tab_v[128 + h, pl.ds(16 * k, 16)] for k in range(4)]

        def right_row(r, c):
            for k in range(4):
                blk_v[r, pl.ds(64 + 16 * k, 16)] = cks[k]
            return c

        lax.fori_loop(0, 128, right_row, 0)
        pltpu.sync_copy(blk_v, F.at[pl.ds(h * 128, 128)])
        return carry

    lax.fori_loop(0, _BPT, build_block, 0)
    plsc.subcore_barrier()

    def fire_gather(g, b):
        idx = tok_v.at[pl.ds(g * _G, _G)]
        pltpu.make_async_copy(F.at[idx], rows.at[b], sins[b]).start()

    def wait_gather(b):
        pltpu.make_async_copy(
            F.at[tok_v.at[pl.ds(0, _G)]], rows.at[b], sins[b]).wait()

    def fire_out(g, b):
        pltpu.make_async_copy(
            rows.at[b], out_hbm.at[pl.ds(base + g * _G, _G)], souts[b]).start()

    def wait_out(b):
        pltpu.make_async_copy(
            rows.at[b], out_hbm.at[pl.ds(base, _G)], souts[b]).wait()

    for b in range(_NBUF):
        fire_gather(b, b)

    def step(g, b):
        wait_gather(b)
        fire_out(g, b)

        @pl.when(g + _NBUF < _NCHUNK)
        def _():
            wait_out(b)
            fire_gather(g + _NBUF, b)

    def round_(p, carry):
        for b in range(_NBUF):
            step(_NBUF * p + b, b)
        return carry

    lax.fori_loop(0, _NSTEP, round_, 0)
    for b in range(_NBUF):
        wait_out(b)


@functools.partial(
    pl.kernel,
    out_type=(
        jax.ShapeDtypeStruct((_N_TOK, 2 * _D), jnp.float32),
        jax.ShapeDtypeStruct((_NC, _V, 2 * _D), jnp.float32),
    ),
    mesh=plsc.VectorSubcoreMesh(core_axis_name="c", subcore_axis_name="s"),
    compiler_params=pltpu.CompilerParams(use_tc_tiling_on_sc=False),
    scratch_types=[
        pltpu.VMEM((_C,), jnp.int32),
        pltpu.VMEM((256, _D), jnp.float32),
        pltpu.VMEM((128, 2 * _D), jnp.float32),
        pltpu.VMEM((_NBUF, _G, 2 * _D), jnp.float32),
        [pltpu.SemaphoreType.DMA] * _NBUF,
        [pltpu.SemaphoreType.DMA] * _NBUF,
    ],
)
def _lookup(tok_hbm, tab_hbm, out_hbm, fus_hbm, *rest):
    _body(tok_hbm, tab_hbm, out_hbm, fus_hbm, *rest)


def kernel(tokens, codebook):
    tok = tokens.astype(jnp.int32).reshape(_N_TOK)
    tab = codebook.reshape(2 * 128, _D)
    out2, _ = _lookup(tok, tab)
    return out2.reshape(_B, _T, 2 * _D)
